# baseline (speedup 1.0000x reference)
import jax
import jax.numpy as jnp
from jax.experimental import pallas as pl
from jax.experimental.pallas import tpu as pltpu


BLOCK_L = 2048


def _add_kernel(x_ref, pos_ref, out_ref):
    out_ref[...] = x_ref[...] + pos_ref[...]


def kernel(x, pos_emb):
    B, L, D = x.shape
    nl = L // BLOCK_L
    return pl.pallas_call(
        _add_kernel,
        grid=(nl, B),
        in_specs=[
            pl.BlockSpec((1, BLOCK_L, D), lambda l, b: (b, l, 0)),
            pl.BlockSpec((BLOCK_L, D), lambda l, b: (l, 0)),
        ],
        out_specs=pl.BlockSpec((1, BLOCK_L, D), lambda l, b: (b, l, 0)),
        out_shape=jax.ShapeDtypeStruct((B, L, D), x.dtype),
        compiler_params=pltpu.CompilerParams(
            dimension_semantics=("parallel", "parallel"),
        ),
    )(x, pos_emb)
